# Initial kernel scaffold; baseline (speedup 1.0000x reference)
import jax
import jax.numpy as jnp
from jax.experimental import pallas as pl


def kernel(feat, coords, W_embede, W_msg, W_coord, W_node, W_fc1, b_fc1, W_fc2, b_fc2, edge_index):
    raise NotImplementedError("write your pallas kernel here")


# speedup vs baseline: 3.8644x; 3.8644x over previous
import functools

import jax
import jax.numpy as jnp
from jax import lax
from jax.experimental import pallas as pl
from jax.experimental.pallas import tpu as pltpu
from jax.experimental.pallas import tpu_sc as plsc

N = 10000
E = 320000
D = 128
H = 128
NPAD = 10240

NC = 2
NS = 16
NW = NC * NS
EPT = E // NW
CC = 80
NCHUNK = EPT // CC
RPT = NPAD // NS
CPAD = 16


def _pre_body(feat_ref, we_ref, wms_ref, wmd_ref, h_ref, a_ref, b_ref):
    f = feat_ref[...]
    h = jnp.dot(f, we_ref[...], preferred_element_type=jnp.float32)
    h_ref[...] = h
    a_ref[...] = jnp.dot(h, wms_ref[...], preferred_element_type=jnp.float32)
    b_ref[...] = jnp.dot(h, wmd_ref[...], preferred_element_type=jnp.float32)


def _edge_body(a_hbm, b_hbm, cp_hbm, src_hbm, dst_hbm, wpk_hbm,
               agg_out, rcd_out,
               srcv, dstv, arows, brows, csrc, cdst, mbuf, rcdbuf, wbuf,
               aggs, rcds):
    cid = lax.axis_index("c")
    sid = lax.axis_index("s")
    w = cid * NS + sid

    pltpu.sync_copy(wpk_hbm, wbuf)
    wd2 = [wbuf[pl.ds(16 * j, 16)] for j in range(8)]
    wc = [wbuf[pl.ds(128 + 16 * j, 16)] for j in range(8)]
    lane = lax.iota(jnp.int32, 16)
    oh3 = jnp.where(lane == 3, 1.0, 0.0).astype(jnp.float32)
    zero16 = jnp.zeros((16,), jnp.float32)

    def _zrow(r, c):
        for j in range(8):
            mbuf[r, pl.ds(16 * j, 16)] = zero16
        rcdbuf[r] = zero16
        return c
    lax.fori_loop(0, CC, _zrow, 0)
    row0 = sid * RPT
    for k in range(RPT // CC):
        pltpu.sync_copy(mbuf, aggs.at[pl.ds(row0 + k * CC, CC)])
        pltpu.sync_copy(rcdbuf, rcds.at[pl.ds(row0 + k * CC, CC)])
    plsc.subcore_barrier()

    e0 = w * EPT

    def _chunk(i, c):
        base = e0 + i * CC
        pltpu.sync_copy(src_hbm.at[pl.ds(base, CC)], srcv)
        pltpu.sync_copy(dst_hbm.at[pl.ds(base, CC)], dstv)
        pltpu.sync_copy(a_hbm.at[srcv], arows)
        pltpu.sync_copy(b_hbm.at[dstv], brows)
        pltpu.sync_copy(cp_hbm.at[srcv], csrc)
        pltpu.sync_copy(cp_hbm.at[dstv], cdst)

        def _edge(e, cc):
            rel = csrc[e] - cdst[e]
            d2 = jnp.sum(rel * rel)
            cwa = zero16
            for j in range(8):
                a = arows[e, pl.ds(16 * j, 16)]
                b = brows[e, pl.ds(16 * j, 16)]
                m = jnp.maximum(a + b + d2 * wd2[j], 0.0)
                mbuf[e, pl.ds(16 * j, 16)] = m
                cwa = cwa + m * wc[j]
            cw = jnp.sum(cwa)
            rcdbuf[e] = rel * cw + oh3
            return cc
        lax.fori_loop(0, CC, _edge, 0)

        pltpu.sync_copy(mbuf, aggs.at[dstv], add=True)
        pltpu.sync_copy(rcdbuf, rcds.at[dstv], add=True)
        return c

    lax.fori_loop(0, NCHUNK, _chunk, 0)
    plsc.subcore_barrier()

    pltpu.sync_copy(aggs.at[pl.ds(row0, RPT)], agg_out.at[cid, pl.ds(row0, RPT)])
    pltpu.sync_copy(rcds.at[pl.ds(row0, RPT)], rcd_out.at[cid, pl.ds(row0, RPT)])


def _post_body(h_ref, agg_ref, rcd_ref, cp_ref, wnh_ref, wna_ref,
               wf1_ref, bf1_ref, wf2_ref, bf2_ref, sc_ref, xf_ref):
    agg = agg_ref[0] + agg_ref[1]
    h = h_ref[...]
    hu = jnp.maximum(
        jnp.dot(h, wnh_ref[...], preferred_element_type=jnp.float32)
        + jnp.dot(agg, wna_ref[...], preferred_element_type=jnp.float32), 0.0)
    z = jnp.maximum(
        jnp.dot(hu, wf1_ref[...], preferred_element_type=jnp.float32)
        + bf1_ref[...], 0.0)
    s = jnp.sum(z * wf2_ref[...], axis=1, keepdims=True) + bf2_ref[...]
    sig = jax.nn.sigmoid(s)
    sc_ref[...] = jnp.broadcast_to(sig, sc_ref.shape)

    rcd = rcd_ref[0] + rcd_ref[1]
    lanes = lax.broadcasted_iota(jnp.int32, rcd.shape, 1)
    deg = jnp.sum(jnp.where(lanes == 3, rcd, 0.0), axis=1, keepdims=True)
    deg = jnp.maximum(deg, 1.0)
    xf_ref[...] = cp_ref[...] + rcd / deg


def _edge_call(a_tab, b_tab, coordsp, src, dst, wpk):
    f32 = jnp.float32
    mesh = plsc.VectorSubcoreMesh(
        core_axis_name="c", subcore_axis_name="s",
        num_cores=NC, num_subcores=NS)
    edge_fn = functools.partial(
        pl.kernel,
        out_type=(jax.ShapeDtypeStruct((NC, NPAD, H), f32),
                  jax.ShapeDtypeStruct((NC, NPAD, CPAD), f32)),
        mesh=mesh,
        scratch_types=[
            pltpu.VMEM((CC,), jnp.int32),
            pltpu.VMEM((CC,), jnp.int32),
            pltpu.VMEM((CC, H), f32),
            pltpu.VMEM((CC, H), f32),
            pltpu.VMEM((CC, CPAD), f32),
            pltpu.VMEM((CC, CPAD), f32),
            pltpu.VMEM((CC, H), f32),
            pltpu.VMEM((CC, CPAD), f32),
            pltpu.VMEM((2 * H,), f32),
            pltpu.VMEM_SHARED((NPAD, H), f32),
            pltpu.VMEM_SHARED((NPAD, CPAD), f32),
        ],
        compiler_params=pltpu.CompilerParams(
            needs_layout_passes=False, use_tc_tiling_on_sc=False),
    )(_edge_body)
    return edge_fn(a_tab, b_tab, coordsp, src, dst, wpk)


def kernel(feat, coords, W_embede, W_msg, W_coord, W_node, W_fc1, b_fc1,
           W_fc2, b_fc2, edge_index):
    f32 = jnp.float32
    wms = W_msg[:D]
    wmd = W_msg[D:2 * D]
    wpk = jnp.concatenate([W_msg[2 * D], W_coord[:, 0]])
    featp = jnp.concatenate([feat, jnp.zeros((NPAD - N, D), f32)], axis=0)
    coordsp = jnp.concatenate(
        [jnp.concatenate([coords, jnp.zeros((N, CPAD - 3), f32)], axis=1),
         jnp.zeros((NPAD - N, CPAD), f32)], axis=0)
    src = edge_index[0]
    dst = edge_index[1]

    bn = 2048
    grid = (NPAD // bn,)
    row_spec = pl.BlockSpec((bn, D), lambda i: (i, 0))
    w_spec = pl.BlockSpec((D, D), lambda i: (0, 0))
    h, a_tab, b_tab = pl.pallas_call(
        _pre_body,
        grid=grid,
        in_specs=[row_spec, w_spec, w_spec, w_spec],
        out_specs=[row_spec, row_spec, row_spec],
        out_shape=[jax.ShapeDtypeStruct((NPAD, D), f32)] * 3,
    )(featp, W_embede, wms, wmd)

    agg2, rcd2 = _edge_call(a_tab, b_tab, coordsp, src, dst, wpk)

    wnh = W_node[:D]
    wna = W_node[D:]
    bf1 = b_fc1.reshape(1, D)
    wf2r = W_fc2.T
    bf2 = b_fc2.reshape(1, 1)
    pad_spec = pl.BlockSpec((bn, CPAD), lambda i: (i, 0))
    sc16, xf16 = pl.pallas_call(
        _post_body,
        grid=grid,
        in_specs=[
            row_spec,
            pl.BlockSpec((NC, bn, H), lambda i: (0, i, 0)),
            pl.BlockSpec((NC, bn, CPAD), lambda i: (0, i, 0)),
            pad_spec,
            w_spec, w_spec, w_spec,
            pl.BlockSpec((1, D), lambda i: (0, 0)),
            pl.BlockSpec((1, D), lambda i: (0, 0)),
            pl.BlockSpec((1, 1), lambda i: (0, 0)),
        ],
        out_specs=[pad_spec, pad_spec],
        out_shape=[jax.ShapeDtypeStruct((NPAD, CPAD), f32)] * 2,
    )(h, agg2, rcd2, coordsp, wnh, wna, W_fc1, bf1, wf2r, bf2)

    return jnp.concatenate([sc16[:N, :1], xf16[:N, :3]], axis=1)


# speedup vs baseline: 13.7912x; 3.5688x over previous
import functools

import jax
import jax.numpy as jnp
from jax import lax
from jax.experimental import pallas as pl
from jax.experimental.pallas import tpu as pltpu
from jax.experimental.pallas import tpu_sc as plsc

N = 10000
E = 320000
D = 128
H = 128
NPAD = 10240

NC = 2
NS = 16
NW = NC * NS
EPT = E // NW
CC = 40
NCHUNK = EPT // CC
KB = 10
NBLK = NCHUNK // KB
RPT = NPAD // NS
CPAD = 16


def _pre_body(feat_ref, we_ref, wms_ref, wmd_ref, h_ref, a_ref, b_ref):
    f = feat_ref[...]
    h = jnp.dot(f, we_ref[...], preferred_element_type=jnp.float32)
    h_ref[...] = h
    a_ref[...] = jnp.dot(h, wms_ref[...], preferred_element_type=jnp.float32)
    b_ref[...] = jnp.dot(h, wmd_ref[...], preferred_element_type=jnp.float32)


def _edge_body(a_hbm, b_hbm, cp_hbm, epk_hbm, wpk_hbm,
               agg_out, rcd_out,
               idxblk, ar0, ar1, br0, br1, cs0, cs1, cd0, cd1,
               mb0, mb1, rb0, rb1, wbuf, aggs, rcds,
               gsem0, gsem1, ssem0, ssem1, isem):
    cid = lax.axis_index("c")
    sid = lax.axis_index("s")
    w = cid * NS + sid

    ar = (ar0, ar1)
    br = (br0, br1)
    cs = (cs0, cs1)
    cd = (cd0, cd1)
    mb = (mb0, mb1)
    rb = (rb0, rb1)
    gsem = (gsem0, gsem1)
    ssem = (ssem0, ssem1)

    pltpu.sync_copy(epk_hbm.at[w, 0], idxblk.at[0])
    pltpu.async_copy(epk_hbm.at[w, 1], idxblk.at[1], isem)
    pltpu.sync_copy(wpk_hbm, wbuf)
    wd2 = [wbuf[pl.ds(16 * j, 16)] for j in range(8)]
    wc = [wbuf[pl.ds(128 + 16 * j, 16)] for j in range(8)]
    lane = lax.iota(jnp.int32, 16)
    oh3 = jnp.where(lane == 3, 1.0, 0.0).astype(jnp.float32)
    zero16 = jnp.zeros((16,), jnp.float32)

    def _zrow(r, c):
        for j in range(8):
            mb0[r, pl.ds(16 * j, 16)] = zero16
        rb0[r] = zero16
        return c
    lax.fori_loop(0, CC, _zrow, 0)
    row0 = sid * RPT
    for k in range(RPT // CC):
        pltpu.sync_copy(mb0, aggs.at[pl.ds(row0 + k * CC, CC)])
        pltpu.sync_copy(rb0, rcds.at[pl.ds(row0 + k * CC, CC)])
    plsc.subcore_barrier()

    def idx_row(i, s):
        blk = i // KB
        return idxblk.at[blk % 2, s, i % KB]

    def issue_gather(i, b):
        pltpu.async_copy(a_hbm.at[idx_row(i, 0)], ar[b], gsem[b])
        pltpu.async_copy(b_hbm.at[idx_row(i, 1)], br[b], gsem[b])
        pltpu.async_copy(cp_hbm.at[idx_row(i, 0)], cs[b], gsem[b])
        pltpu.async_copy(cp_hbm.at[idx_row(i, 1)], cd[b], gsem[b])

    def wait_gather(b):
        pltpu.make_async_copy(a_hbm.at[pl.ds(0, CC)], ar[b], gsem[b]).wait()
        pltpu.make_async_copy(b_hbm.at[pl.ds(0, CC)], br[b], gsem[b]).wait()
        pltpu.make_async_copy(cp_hbm.at[pl.ds(0, CC)], cs[b], gsem[b]).wait()
        pltpu.make_async_copy(cp_hbm.at[pl.ds(0, CC)], cd[b], gsem[b]).wait()

    def issue_scatter(i, b):
        pltpu.async_copy(mb[b], aggs.at[idx_row(i, 1)], ssem[b], add=True)
        pltpu.async_copy(rb[b], rcds.at[idx_row(i, 1)], ssem[b], add=True)

    def wait_scatter(b):
        pltpu.make_async_copy(a_hbm.at[pl.ds(0, CC)], mb[b], ssem[b]).wait()
        pltpu.make_async_copy(cp_hbm.at[pl.ds(0, CC)], rb[b], ssem[b]).wait()

    def compute(b):
        arb, brb, csb, cdb, mbb, rbb = ar[b], br[b], cs[b], cd[b], mb[b], rb[b]

        @plsc.parallel_loop(0, CC, unroll=2)
        def _edge(e):
            rel = csb[e] - cdb[e]
            d2 = jnp.sum(rel * rel)
            cwa = zero16
            for j in range(8):
                a = arb[e, pl.ds(16 * j, 16)]
                bb = brb[e, pl.ds(16 * j, 16)]
                m = jnp.maximum(a + bb + d2 * wd2[j], 0.0)
                mbb[e, pl.ds(16 * j, 16)] = m
                cwa = cwa + m * wc[j]
            cw = jnp.sum(cwa)
            rbb[e] = rel * cw + oh3

    issue_gather(0, 0)

    def _pair(k, c):
        for b in range(2):
            i = 2 * k + b

            @pl.when(jnp.logical_and(i % KB == 8, i < (NBLK - 1) * KB))
            def _():
                pltpu.make_async_copy(
                    epk_hbm.at[w, 0], idxblk.at[0], isem).wait()

            @pl.when(i + 1 < NCHUNK)
            def _():
                issue_gather(i + 1, 1 - b)

            @pl.when(jnp.logical_and(i % KB == 2,
                                     jnp.logical_and(i >= KB + 2,
                                                     i < (NBLK - 1) * KB + 2)))
            def _():
                blk = i // KB + 1
                pltpu.async_copy(epk_hbm.at[w, blk], idxblk.at[blk % 2], isem)

            wait_gather(b)

            @pl.when(i >= 2)
            def _():
                wait_scatter(b)

            compute(b)
            issue_scatter(i, b)
        return c

    lax.fori_loop(0, NCHUNK // 2, _pair, 0)
    wait_scatter(0)
    wait_scatter(1)
    plsc.subcore_barrier()

    pltpu.sync_copy(aggs.at[pl.ds(row0, RPT)], agg_out.at[cid, pl.ds(row0, RPT)])
    pltpu.sync_copy(rcds.at[pl.ds(row0, RPT)], rcd_out.at[cid, pl.ds(row0, RPT)])


def _post_body(h_ref, agg_ref, rcd_ref, cp_ref, wnh_ref, wna_ref,
               wf1_ref, bf1_ref, wf2_ref, bf2_ref, sc_ref, xf_ref):
    agg = agg_ref[0] + agg_ref[1]
    h = h_ref[...]
    hu = jnp.maximum(
        jnp.dot(h, wnh_ref[...], preferred_element_type=jnp.float32)
        + jnp.dot(agg, wna_ref[...], preferred_element_type=jnp.float32), 0.0)
    z = jnp.maximum(
        jnp.dot(hu, wf1_ref[...], preferred_element_type=jnp.float32)
        + bf1_ref[...], 0.0)
    s = jnp.sum(z * wf2_ref[...], axis=1, keepdims=True) + bf2_ref[...]
    sig = jax.nn.sigmoid(s)
    sc_ref[...] = jnp.broadcast_to(sig, sc_ref.shape)

    rcd = rcd_ref[0] + rcd_ref[1]
    lanes = lax.broadcasted_iota(jnp.int32, rcd.shape, 1)
    deg = jnp.sum(jnp.where(lanes == 3, rcd, 0.0), axis=1, keepdims=True)
    deg = jnp.maximum(deg, 1.0)
    xf_ref[...] = cp_ref[...] + rcd / deg


def _edge_call(a_tab, b_tab, coordsp, epk, wpk):
    f32 = jnp.float32
    mesh = plsc.VectorSubcoreMesh(
        core_axis_name="c", subcore_axis_name="s",
        num_cores=NC, num_subcores=NS)
    edge_fn = functools.partial(
        pl.kernel,
        out_type=(jax.ShapeDtypeStruct((NC, NPAD, H), f32),
                  jax.ShapeDtypeStruct((NC, NPAD, CPAD), f32)),
        mesh=mesh,
        scratch_types=[
            pltpu.VMEM((2, 2, KB, CC), jnp.int32),
            pltpu.VMEM((CC, H), f32),
            pltpu.VMEM((CC, H), f32),
            pltpu.VMEM((CC, H), f32),
            pltpu.VMEM((CC, H), f32),
            pltpu.VMEM((CC, CPAD), f32),
            pltpu.VMEM((CC, CPAD), f32),
            pltpu.VMEM((CC, CPAD), f32),
            pltpu.VMEM((CC, CPAD), f32),
            pltpu.VMEM((CC, H), f32),
            pltpu.VMEM((CC, H), f32),
            pltpu.VMEM((CC, CPAD), f32),
            pltpu.VMEM((CC, CPAD), f32),
            pltpu.VMEM((2 * H,), f32),
            pltpu.VMEM_SHARED((NPAD, H), f32),
            pltpu.VMEM_SHARED((NPAD, CPAD), f32),
            pltpu.SemaphoreType.DMA,
            pltpu.SemaphoreType.DMA,
            pltpu.SemaphoreType.DMA,
            pltpu.SemaphoreType.DMA,
            pltpu.SemaphoreType.DMA,
        ],
        compiler_params=pltpu.CompilerParams(
            needs_layout_passes=False, use_tc_tiling_on_sc=False),
    )(_edge_body)
    return edge_fn(a_tab, b_tab, coordsp, epk, wpk)


def kernel(feat, coords, W_embede, W_msg, W_coord, W_node, W_fc1, b_fc1,
           W_fc2, b_fc2, edge_index):
    f32 = jnp.float32
    wms = W_msg[:D]
    wmd = W_msg[D:2 * D]
    wpk = jnp.concatenate([W_msg[2 * D], W_coord[:, 0]])
    featp = jnp.concatenate([feat, jnp.zeros((NPAD - N, D), f32)], axis=0)
    coordsp = jnp.concatenate(
        [jnp.concatenate([coords, jnp.zeros((N, CPAD - 3), f32)], axis=1),
         jnp.zeros((NPAD - N, CPAD), f32)], axis=0)
    epk = jnp.stack([edge_index[0].reshape(NW, NBLK, KB, CC),
                     edge_index[1].reshape(NW, NBLK, KB, CC)], axis=2)

    bn = 2048
    grid = (NPAD // bn,)
    row_spec = pl.BlockSpec((bn, D), lambda i: (i, 0))
    w_spec = pl.BlockSpec((D, D), lambda i: (0, 0))
    h, a_tab, b_tab = pl.pallas_call(
        _pre_body,
        grid=grid,
        in_specs=[row_spec, w_spec, w_spec, w_spec],
        out_specs=[row_spec, row_spec, row_spec],
        out_shape=[jax.ShapeDtypeStruct((NPAD, D), f32)] * 3,
    )(featp, W_embede, wms, wmd)

    agg2, rcd2 = _edge_call(a_tab, b_tab, coordsp, epk, wpk)

    wnh = W_node[:D]
    wna = W_node[D:]
    bf1 = b_fc1.reshape(1, D)
    wf2r = W_fc2.T
    bf2 = b_fc2.reshape(1, 1)
    pad_spec = pl.BlockSpec((bn, CPAD), lambda i: (i, 0))
    sc16, xf16 = pl.pallas_call(
        _post_body,
        grid=grid,
        in_specs=[
            row_spec,
            pl.BlockSpec((NC, bn, H), lambda i: (0, i, 0)),
            pl.BlockSpec((NC, bn, CPAD), lambda i: (0, i, 0)),
            pad_spec,
            w_spec, w_spec, w_spec,
            pl.BlockSpec((1, D), lambda i: (0, 0)),
            pl.BlockSpec((1, D), lambda i: (0, 0)),
            pl.BlockSpec((1, 1), lambda i: (0, 0)),
        ],
        out_specs=[pad_spec, pad_spec],
        out_shape=[jax.ShapeDtypeStruct((NPAD, CPAD), f32)] * 2,
    )(h, agg2, rcd2, coordsp, wnh, wna, W_fc1, bf1, wf2r, bf2)

    return jnp.concatenate([sc16[:N, :1], xf16[:N, :3]], axis=1)
